# baseline (speedup 1.0000x reference)
import jax
import jax.numpy as jnp
from jax.experimental import pallas as pl

NUM_CLASSES = 1000
PACK = 16
ROWS = 4096 * 20 // PACK
COLS = PACK * NUM_CLASSES
BLOCK = 40


def _onehot_body(cls_ref, exp_ref, pos_ref, out_ref):
    m = jnp.dot(cls_ref[...], exp_ref[...],
                preferred_element_type=jnp.float32)
    out_ref[...] = (m == pos_ref[...]).astype(jnp.int32)


def kernel(classes):
    cls_f = classes.reshape(ROWS, PACK).astype(jnp.float32)
    t = jnp.arange(COLS, dtype=jnp.int32) // NUM_CLASSES
    expander = (t[None, :] == jnp.arange(PACK, dtype=jnp.int32)[:, None]
                ).astype(jnp.float32)
    pos = (jnp.arange(COLS, dtype=jnp.int32) % NUM_CLASSES
           ).astype(jnp.float32)[None, :]
    out = pl.pallas_call(
        _onehot_body,
        grid=(ROWS // BLOCK,),
        in_specs=[
            pl.BlockSpec((BLOCK, PACK), lambda i: (i, 0)),
            pl.BlockSpec((PACK, COLS), lambda i: (0, 0)),
            pl.BlockSpec((1, COLS), lambda i: (0, 0)),
        ],
        out_specs=pl.BlockSpec((BLOCK, COLS), lambda i: (i, 0)),
        out_shape=jax.ShapeDtypeStruct((ROWS, COLS), jnp.int32),
    )(cls_f, expander, pos)
    return out.reshape(4096, 20, NUM_CLASSES)


# speedup vs baseline: 1.0307x; 1.0307x over previous
import functools

import jax
import jax.numpy as jnp
from jax import lax
from jax.experimental import pallas as pl
from jax.experimental.pallas import tpu as pltpu
from jax.experimental.pallas import tpu_sc as plsc

NUM_CLASSES = 1000
N_PLANES = 4096
PLANE_ROWS = 20
NW = 32
PLANES_PER_W = N_PLANES // NW
NB = 4
ROWS_PER_BATCH = NB * PLANE_ROWS
GROUPS = ROWS_PER_BATCH // 16
BATCHES = PLANES_PER_W // NB

_mesh = plsc.VectorSubcoreMesh(core_axis_name="c", subcore_axis_name="s")


@functools.partial(
    pl.kernel,
    mesh=_mesh,
    compiler_params=pltpu.CompilerParams(use_tc_tiling_on_sc=False),
    out_type=jax.ShapeDtypeStruct((N_PLANES, PLANE_ROWS, NUM_CLASSES), jnp.int32),
    scratch_types=[
        pltpu.VMEM((PLANES_PER_W * PLANE_ROWS,), jnp.int32),
        pltpu.VMEM((NB, PLANE_ROWS, NUM_CLASSES), jnp.int32),
    ],
)
def _sc_onehot(cls_hbm, zeros_hbm, out_hbm, cls_v, buf):
    wid = lax.axis_index("s") * 2 + lax.axis_index("c")
    plane0 = wid * PLANES_PER_W
    pltpu.sync_copy(cls_hbm.at[pl.ds(plane0 * PLANE_ROWS, PLANES_PER_W * PLANE_ROWS)],
                    cls_v)
    pltpu.sync_copy(zeros_hbm, buf)

    lanes = lax.iota(jnp.int32, 16)

    def scatter_batch(i, val):
        loc_lanes = lax.iota(jnp.int32, 16)
        for g in range(GROUPS):
            r0 = i * ROWS_PER_BATCH + g * 16
            cls16 = cls_v[pl.ds(r0, 16)]
            for k in range(16):
                r = g * 16 + k
                p, j = divmod(r, PLANE_ROWS)
                c = cls16[k]
                cb = jnp.minimum(c & ~15, NUM_CLASSES - 16)
                d = jnp.minimum(jnp.abs(loc_lanes - (c - cb)), 31)
                vec = (jnp.int32(1) >> d) * val
                buf[p, j, pl.ds(cb, 16)] = vec

    def batch(i, carry):
        scatter_batch(i, jnp.int32(1))
        pltpu.sync_copy(buf, out_hbm.at[pl.ds(plane0 + i * NB, NB)])
        scatter_batch(i, jnp.int32(0))
        return carry

    lax.fori_loop(0, BATCHES, batch, 0)


def kernel(classes):
    cls_flat = classes.reshape(-1)
    zeros_help = jnp.zeros((NB, PLANE_ROWS, NUM_CLASSES), jnp.int32)
    return _sc_onehot(cls_flat, zeros_help)


# speedup vs baseline: 1.5155x; 1.4703x over previous
import functools

import jax
import jax.numpy as jnp
from jax import lax
from jax.experimental import pallas as pl
from jax.experimental.pallas import tpu as pltpu
from jax.experimental.pallas import tpu_sc as plsc

NUM_CLASSES = 1000
N_PLANES = 4096
PLANE_ROWS = 20
NW = 32
PLANES_PER_W = N_PLANES // NW
NB = 4
ROWS_PER_BATCH = NB * PLANE_ROWS
GROUPS = ROWS_PER_BATCH // 16
BATCHES = PLANES_PER_W // NB

_mesh = plsc.VectorSubcoreMesh(core_axis_name="c", subcore_axis_name="s")


@functools.partial(
    pl.kernel,
    mesh=_mesh,
    compiler_params=pltpu.CompilerParams(use_tc_tiling_on_sc=True),
    out_type=jax.ShapeDtypeStruct((N_PLANES, PLANE_ROWS, NUM_CLASSES), jnp.int32),
    scratch_types=[
        pltpu.VMEM((PLANES_PER_W * PLANE_ROWS,), jnp.int32),
        pltpu.VMEM((NB, PLANE_ROWS, NUM_CLASSES), jnp.int32),
    ],
)
def _sc_onehot(cls_hbm, zeros_hbm, out_hbm, cls_v, buf):
    wid = lax.axis_index("s") * 2 + lax.axis_index("c")
    plane0 = wid * PLANES_PER_W
    pltpu.sync_copy(cls_hbm.at[pl.ds(plane0 * PLANE_ROWS, PLANES_PER_W * PLANE_ROWS)],
                    cls_v)
    pltpu.sync_copy(zeros_hbm, buf)

    lanes = lax.iota(jnp.int32, 16)

    def scatter_batch(i, val):
        loc_lanes = lax.iota(jnp.int32, 16)
        for g in range(GROUPS):
            r0 = i * ROWS_PER_BATCH + g * 16
            cls16 = cls_v[pl.ds(r0, 16)]
            for k in range(16):
                r = g * 16 + k
                p, j = divmod(r, PLANE_ROWS)
                c = cls16[k]
                cb = pl.multiple_of(c & ~15, 16)
                d = jnp.minimum(jnp.abs(loc_lanes - (c - cb)), 31)
                vec = (jnp.int32(1) >> d) * val
                buf[p, j, pl.ds(cb, 16)] = vec

    def batch(i, carry):
        scatter_batch(i, jnp.int32(1))
        pltpu.sync_copy(buf, out_hbm.at[pl.ds(plane0 + i * NB, NB)])
        scatter_batch(i, jnp.int32(0))
        return carry

    lax.fori_loop(0, BATCHES, batch, 0)


def kernel(classes):
    cls_flat = classes.reshape(-1)
    zeros_help = jnp.zeros((NB, PLANE_ROWS, NUM_CLASSES), jnp.int32)
    return _sc_onehot(cls_flat, zeros_help)
